# Initial kernel scaffold; baseline (speedup 1.0000x reference)
import jax
import jax.numpy as jnp
from jax.experimental import pallas as pl


def kernel(x_pointwise, sphere_points, batch, natoms, W1s, b1s, W2s, b2s, W1i, b1i, W2i, b2i):
    raise NotImplementedError("write your pallas kernel here")


# speedup vs baseline: 1.1966x; 1.1966x over previous
import functools
import numpy as np
import jax
import jax.numpy as jnp
from jax.experimental import pallas as pl


def _mlp_segsum_kernel(x_ref, w1_ref, b1_ref, w2_ref, b2_ref, c1_ref, oh_ref,
                       batch_ref, acc_ref, *, bn, s, g):
    i = pl.program_id(0)

    x = x_ref[...]
    h = jnp.dot(x, w1_ref[...], preferred_element_type=jnp.float32)
    h = h + b1_ref[...]
    h = h * jax.nn.sigmoid(h)
    r = jnp.dot(h, w2_ref[...], preferred_element_type=jnp.float32)
    r = r + b2_ref[...]

    col = jax.lax.broadcasted_iota(jnp.int32, (bn * s, 8), 1)
    inv_s = 1.0 / s
    e = (jnp.where(col == 0, r[:, 0:1] * inv_s, 0.0)
         + r[:, 1:2] * c1_ref[...]
         + jnp.where(col == 6, inv_s, 0.0))

    node = jnp.dot(oh_ref[...], e, preferred_element_type=jnp.float32)

    seg = batch_ref[0, 0, :]
    gids = jax.lax.broadcasted_iota(jnp.int32, (g, bn), 0)
    m = (gids == seg[None, :]).astype(jnp.float32)
    contrib = jnp.dot(m, node, preferred_element_type=jnp.float32)

    @pl.when(i == 0)
    def _init():
        acc_ref[...] = jnp.zeros_like(acc_ref)

    acc_ref[...] += contrib


def kernel(x_pointwise, sphere_points, batch, natoms, W1s, b1s, W2s, b2s,
           W1i, b1i, W2i, b2i):
    N, S, D = x_pointwise.shape
    G = natoms.shape[0]
    BN = 64
    num_blocks = N // BN

    pts = sphere_points / jnp.linalg.norm(sphere_points, axis=-1, keepdims=True)
    x_, y_, z_ = pts[:, 0], pts[:, 1], pts[:, 2]
    s15 = 15.0 ** 0.5
    s5 = 5.0 ** 0.5
    sph = jnp.stack([
        s15 * x_ * z_,
        s15 * x_ * y_,
        s5 * (y_ ** 2 - 0.5 * (x_ ** 2 + z_ ** 2)),
        s15 * y_ * z_,
        (s15 / 2.0) * (z_ ** 2 - x_ ** 2),
    ], axis=-1) / (4.0 * np.pi) ** 0.5

    w1 = jnp.concatenate([W1s.T, W1i.T], axis=1)
    b1 = jnp.concatenate([b1s, b1i]).reshape(1, 2 * D)
    w2 = jnp.zeros((2 * D, 8), jnp.float32)
    w2 = w2.at[:D, 0].set(W2s[0])
    w2 = w2.at[D:, 1].set(W2i[0])
    b2 = jnp.zeros((1, 8), jnp.float32)
    b2 = b2.at[0, 0].set(b2s[0])
    b2 = b2.at[0, 1].set(b2i[0])
    c1_one = jnp.zeros((S, 8), jnp.float32).at[:, 1:6].set(sph / S)
    c1 = jnp.tile(c1_one, (BN, 1))
    oh = jnp.asarray(np.repeat(np.eye(BN, dtype=np.float32), S, axis=1))

    batch_i32 = batch.astype(jnp.int32).reshape(num_blocks, 1, BN)
    x2d = x_pointwise.reshape(N * S, D)

    acc = pl.pallas_call(
        functools.partial(_mlp_segsum_kernel, bn=BN, s=S, g=G),
        grid=(num_blocks,),
        in_specs=[
            pl.BlockSpec((BN * S, D), lambda i: (i, 0)),
            pl.BlockSpec((D, 2 * D), lambda i: (0, 0)),
            pl.BlockSpec((1, 2 * D), lambda i: (0, 0)),
            pl.BlockSpec((2 * D, 8), lambda i: (0, 0)),
            pl.BlockSpec((1, 8), lambda i: (0, 0)),
            pl.BlockSpec((BN * S, 8), lambda i: (0, 0)),
            pl.BlockSpec((BN, BN * S), lambda i: (0, 0)),
            pl.BlockSpec((1, 1, BN), lambda i: (i, 0, 0)),
        ],
        out_specs=pl.BlockSpec((G, 8), lambda i: (0, 0)),
        out_shape=jax.ShapeDtypeStruct((G, 8), jnp.float32),
    )(x2d, w1, b1, w2, b2, c1, oh, batch_i32)

    counts = jnp.maximum(acc[:, 6], 1.0)
    scalar = acc[:, 0] / counts
    irrep2 = acc[:, 1:6] / counts[:, None]
    return (scalar, irrep2)
